# baseline (speedup 1.0000x reference)
import functools

import jax
import jax.numpy as jnp
from jax.experimental import pallas as pl


def _moe_kernel(x_ref, wg_ref, bg_ref, w1_ref, b1_ref, w2_ref, b2_ref, o_ref):
    j = pl.program_id(1)
    xb = x_ref[...]
    logits = jnp.dot(xb, wg_ref[...], preferred_element_type=jnp.float32)
    logits = logits + bg_ref[...]
    e = logits.shape[-1]
    v1 = jnp.max(logits, axis=-1, keepdims=True)
    iota = jax.lax.broadcasted_iota(jnp.int32, logits.shape, 1)
    idx1 = jnp.min(jnp.where(logits >= v1, iota, e), axis=-1, keepdims=True)
    v2 = jnp.max(jnp.where(iota == idx1, -jnp.inf, logits), axis=-1, keepdims=True)
    scale = 2.0 - 1.0 / (1.0 + jnp.exp(v2 - v1))
    h1 = jnp.dot(xb.astype(jnp.bfloat16), w1_ref[...].astype(jnp.bfloat16),
                 preferred_element_type=jnp.float32)
    h1 = jnp.maximum(h1 + b1_ref[...], 0.0)
    p = jnp.dot(h1.astype(jnp.bfloat16), w2_ref[...].astype(jnp.bfloat16),
                preferred_element_type=jnp.float32)

    @pl.when(j == 0)
    def _():
        o_ref[...] = scale * (p + b2_ref[...])

    @pl.when(j > 0)
    def _():
        o_ref[...] += scale * p


@functools.partial(jax.jit, static_argnames=())
def kernel(x, Wg, bg, W1, b1, W2, b2):
    B, N, D = x.shape
    T = B * N
    E = Wg.shape[1]
    H = W1.shape[1]
    BT = 2048
    CH = 512
    xf = x.reshape(T, D)

    out = pl.pallas_call(
        _moe_kernel,
        grid=(T // BT, H // CH),
        in_specs=[
            pl.BlockSpec((BT, D), lambda i, j: (i, 0)),
            pl.BlockSpec((D, E), lambda i, j: (0, 0)),
            pl.BlockSpec((1, E), lambda i, j: (0, 0)),
            pl.BlockSpec((D, CH), lambda i, j: (0, j)),
            pl.BlockSpec((1, CH), lambda i, j: (0, j)),
            pl.BlockSpec((CH, D), lambda i, j: (j, 0)),
            pl.BlockSpec((1, D), lambda i, j: (0, 0)),
        ],
        out_specs=pl.BlockSpec((BT, D), lambda i, j: (i, 0)),
        out_shape=jax.ShapeDtypeStruct((T, D), x.dtype),
    )(xf, Wg, bg.reshape(1, E), W1, b1.reshape(1, H), W2, b2.reshape(1, D))
    return out.reshape(B, N, D)


# speedup vs baseline: 1.4091x; 1.4091x over previous
import functools

import jax
import jax.numpy as jnp
from jax.experimental import pallas as pl
from jax.experimental.pallas import tpu as pltpu

_NQ = 4


def _moe_kernel(x_ref, wg_ref, bg_ref, b1_ref, b2_ref, w1_hbm, w2_hbm,
                o_ref, w1_v, w2_v, sems):
    i = pl.program_id(0)
    hdim = w1_v.shape[1]
    ck = hdim // _NQ

    @pl.when(i == 0)
    def _():
        for k in range(_NQ):
            sl = pl.ds(k * ck, ck)
            pltpu.make_async_copy(w1_hbm.at[:, sl], w1_v.at[:, sl],
                                  sems.at[2 * k]).start()
            pltpu.make_async_copy(w2_hbm.at[sl, :], w2_v.at[sl, :],
                                  sems.at[2 * k + 1]).start()

    xb = x_ref[...]
    logits = jnp.dot(xb, wg_ref[...], preferred_element_type=jnp.float32)
    logits = logits + bg_ref[...]
    e = logits.shape[-1]
    v1 = jnp.max(logits, axis=-1, keepdims=True)
    iota = jax.lax.broadcasted_iota(jnp.int32, logits.shape, 1)
    idx1 = jnp.min(jnp.where(logits >= v1, iota, e), axis=-1, keepdims=True)
    v2 = jnp.max(jnp.where(iota == idx1, -jnp.inf, logits), axis=-1, keepdims=True)
    scale = 2.0 - 1.0 / (1.0 + jnp.exp(v2 - v1))

    xb16 = xb.astype(jnp.bfloat16)
    h = b2_ref[...]
    for k in range(_NQ):
        sl = pl.ds(k * ck, ck)

        @pl.when(i == 0)
        def _():
            pltpu.make_async_copy(w1_hbm.at[:, sl], w1_v.at[:, sl],
                                  sems.at[2 * k]).wait()

        h1 = jnp.dot(xb16, w1_v[:, sl].astype(jnp.bfloat16),
                     preferred_element_type=jnp.float32)
        h1 = jnp.maximum(h1 + b1_ref[:, sl], 0.0)

        @pl.when(i == 0)
        def _():
            pltpu.make_async_copy(w2_hbm.at[sl, :], w2_v.at[sl, :],
                                  sems.at[2 * k + 1]).wait()

        h = h + jnp.dot(h1.astype(jnp.bfloat16), w2_v[sl, :].astype(jnp.bfloat16),
                        preferred_element_type=jnp.float32)
    o_ref[...] = scale * h


@functools.partial(jax.jit, static_argnames=())
def kernel(x, Wg, bg, W1, b1, W2, b2):
    B, N, D = x.shape
    T = B * N
    E = Wg.shape[1]
    H = W1.shape[1]
    BT = 512
    xf = x.reshape(T, D)

    out = pl.pallas_call(
        _moe_kernel,
        grid=(T // BT,),
        in_specs=[
            pl.BlockSpec((BT, D), lambda i: (i, 0)),
            pl.BlockSpec((D, E), lambda i: (0, 0)),
            pl.BlockSpec((1, E), lambda i: (0, 0)),
            pl.BlockSpec((1, H), lambda i: (0, 0)),
            pl.BlockSpec((1, D), lambda i: (0, 0)),
            pl.BlockSpec(memory_space=pltpu.MemorySpace.HBM),
            pl.BlockSpec(memory_space=pltpu.MemorySpace.HBM),
        ],
        out_specs=pl.BlockSpec((BT, D), lambda i: (i, 0)),
        out_shape=jax.ShapeDtypeStruct((T, D), x.dtype),
        scratch_shapes=[
            pltpu.VMEM((D, H), jnp.float32),
            pltpu.VMEM((H, D), jnp.float32),
            pltpu.SemaphoreType.DMA((2 * _NQ,)),
        ],
    )(xf, Wg, bg.reshape(1, E), b1.reshape(1, H), b2.reshape(1, D), W1, W2)
    return out.reshape(B, N, D)


# speedup vs baseline: 1.5582x; 1.1059x over previous
import functools

import jax
import jax.numpy as jnp
from jax.experimental import pallas as pl


def _moe_kernel(x_ref, wg_ref, bg_ref, w1_ref, b1_ref, w2_ref, b2_ref, o_ref):
    xb = x_ref[...]
    logits = jnp.dot(xb, wg_ref[...], preferred_element_type=jnp.float32)
    logits = logits + bg_ref[...]
    e = logits.shape[-1]
    v1 = jnp.max(logits, axis=-1, keepdims=True)
    iota = jax.lax.broadcasted_iota(jnp.int32, logits.shape, 1)
    idx1 = jnp.min(jnp.where(logits >= v1, iota, e), axis=-1, keepdims=True)
    v2 = jnp.max(jnp.where(iota == idx1, -jnp.inf, logits), axis=-1, keepdims=True)
    scale = 2.0 - 1.0 / (1.0 + jnp.exp(v2 - v1))
    h1 = jnp.dot(xb.astype(jnp.bfloat16), w1_ref[...].astype(jnp.bfloat16),
                 preferred_element_type=jnp.float32)
    h1 = jnp.maximum(h1 + b1_ref[...], 0.0)
    h1 = h1.astype(jnp.bfloat16)
    hdim = h1.shape[1]
    ck = hdim // 4
    parts = []
    for k in range(4):
        parts.append(jnp.dot(h1[:, k * ck:(k + 1) * ck],
                             w2_ref[k * ck:(k + 1) * ck, :].astype(jnp.bfloat16),
                             preferred_element_type=jnp.float32))
    h = (parts[0] + parts[1]) + (parts[2] + parts[3])
    o_ref[...] = scale * (h + b2_ref[...])


@functools.partial(jax.jit, static_argnames=())
def kernel(x, Wg, bg, W1, b1, W2, b2):
    B, N, D = x.shape
    T = B * N
    E = Wg.shape[1]
    H = W1.shape[1]
    BT = 512
    xf = x.reshape(T, D)

    out = pl.pallas_call(
        _moe_kernel,
        grid=(T // BT,),
        in_specs=[
            pl.BlockSpec((BT, D), lambda i: (i, 0)),
            pl.BlockSpec((D, E), lambda i: (0, 0)),
            pl.BlockSpec((1, E), lambda i: (0, 0)),
            pl.BlockSpec((D, H), lambda i: (0, 0)),
            pl.BlockSpec((1, H), lambda i: (0, 0)),
            pl.BlockSpec((H, D), lambda i: (0, 0)),
            pl.BlockSpec((1, D), lambda i: (0, 0)),
        ],
        out_specs=pl.BlockSpec((BT, D), lambda i: (i, 0)),
        out_shape=jax.ShapeDtypeStruct((T, D), x.dtype),
    )(xf, Wg, bg.reshape(1, E), W1, b1.reshape(1, H), W2, b2.reshape(1, D))
    return out.reshape(B, N, D)


# speedup vs baseline: 1.5594x; 1.0008x over previous
import functools

import jax
import jax.numpy as jnp
from jax.experimental import pallas as pl
from jax.experimental.pallas import tpu as pltpu


def _moe_kernel(x_ref, wg_ref, bg_ref, w1_ref, b1_ref, w2_ref, b2_ref, o_ref):
    xb = x_ref[...]
    logits = jnp.dot(xb, wg_ref[...], preferred_element_type=jnp.float32)
    logits = logits + bg_ref[...]
    e = logits.shape[-1]
    v1 = jnp.max(logits, axis=-1, keepdims=True)
    iota = jax.lax.broadcasted_iota(jnp.int32, logits.shape, 1)
    idx1 = jnp.min(jnp.where(logits >= v1, iota, e), axis=-1, keepdims=True)
    v2 = jnp.max(jnp.where(iota == idx1, -jnp.inf, logits), axis=-1, keepdims=True)
    scale = 2.0 - 1.0 / (1.0 + jnp.exp(v2 - v1))
    h1 = jnp.dot(xb.astype(jnp.bfloat16), w1_ref[...].astype(jnp.bfloat16),
                 preferred_element_type=jnp.float32)
    h1 = jnp.maximum(h1 + b1_ref[...], 0.0)
    h1 = h1.astype(jnp.bfloat16)
    hdim = h1.shape[1]
    ck = hdim // 4
    parts = []
    for k in range(4):
        parts.append(jnp.dot(h1[:, k * ck:(k + 1) * ck],
                             w2_ref[k * ck:(k + 1) * ck, :].astype(jnp.bfloat16),
                             preferred_element_type=jnp.float32))
    h = (parts[0] + parts[1]) + (parts[2] + parts[3])
    o_ref[...] = scale * (h + b2_ref[...])


@functools.partial(jax.jit, static_argnames=())
def kernel(x, Wg, bg, W1, b1, W2, b2):
    B, N, D = x.shape
    T = B * N
    E = Wg.shape[1]
    H = W1.shape[1]
    BT = 512
    xf = x.reshape(T, D)

    out = pl.pallas_call(
        _moe_kernel,
        grid=(T // BT,),
        in_specs=[
            pl.BlockSpec((BT, D), lambda i: (i, 0)),
            pl.BlockSpec((D, E), lambda i: (0, 0)),
            pl.BlockSpec((1, E), lambda i: (0, 0)),
            pl.BlockSpec((D, H), lambda i: (0, 0)),
            pl.BlockSpec((1, H), lambda i: (0, 0)),
            pl.BlockSpec((H, D), lambda i: (0, 0)),
            pl.BlockSpec((1, D), lambda i: (0, 0)),
        ],
        out_specs=pl.BlockSpec((BT, D), lambda i: (i, 0)),
        out_shape=jax.ShapeDtypeStruct((T, D), x.dtype),
        compiler_params=pltpu.CompilerParams(
            dimension_semantics=(pltpu.GridDimensionSemantics.PARALLEL,)),
    )(xf, Wg, bg.reshape(1, E), W1, b1.reshape(1, H), W2, b2.reshape(1, D))
    return out.reshape(B, N, D)
